# baseline (speedup 1.0000x reference)
import dataclasses
import functools
import math

import jax
import jax.numpy as jnp
from jax import lax
from jax.experimental import pallas as pl
from jax.experimental.pallas import tpu as pltpu
from jax.experimental.pallas import tpu_sc as plsc

N = 10000
E = 320000
C = 128

NC = 2
NS = 16
L = 16
NW = NC * NS

NP = 10240
CH = 128
EP = 327680
EW = EP // NW
NCHUNK = EW // CH
ROWS_PER_SUB = NP // NS
CHB = 64
NCHB = EW // CHB

INV_SQRT_C = 1.0 / math.sqrt(C)

_mesh = plsc.VectorSubcoreMesh(
    core_axis_name="c", subcore_axis_name="s", num_cores=NC, num_subcores=NS)

_sc_params = pltpu.CompilerParams(
    needs_layout_passes=False, use_tc_tiling_on_sc=False)


def _proj_kernel(x_ref, wq_ref, bq_ref, wk_ref, bk_ref, wv_ref, bv_ref,
                 ws_ref, bs_ref, q_ref, k_ref, v_ref, s_ref):
    x = x_ref[...]
    q_ref[...] = jnp.dot(x, wq_ref[...], preferred_element_type=jnp.float32) + bq_ref[...]
    k_ref[...] = jnp.dot(x, wk_ref[...], preferred_element_type=jnp.float32) + bk_ref[...]
    v_ref[...] = jnp.dot(x, wv_ref[...], preferred_element_type=jnp.float32) + bv_ref[...]
    s_ref[...] = jnp.dot(x, ws_ref[...], preferred_element_type=jnp.float32) + bs_ref[...]


def _sc_alpha_kernel(q_hbm, k_hbm, src_hbm, dst_hbm, alpha_hbm, dpart_hbm,
                     src_v, dst_v, a_v, q_v, q_v1, k_v, k_v1, arow_v, arow_v1,
                     si0, si1, di0, di1, z_v, dacc, gq0, gq1, gk0, gk1):
    cid = lax.axis_index("c")
    sid = lax.axis_index("s")
    wid = sid * NC + cid

    @pl.loop(0, CH)
    def _(i):
        arow_v[i] = jnp.zeros((L,), jnp.float32)
        arow_v1[i] = jnp.zeros((L,), jnp.float32)

    @pl.loop(0, ROWS_PER_SUB)
    def _(i):
        z_v[i] = jnp.zeros((L,), jnp.float32)

    pltpu.sync_copy(z_v, dacc.at[pl.ds(sid * ROWS_PER_SUB, ROWS_PER_SUB)])
    plsc.subcore_barrier()

    pltpu.sync_copy(src_hbm.at[wid], src_v)
    pltpu.sync_copy(dst_hbm.at[wid], dst_v)

    iota = lax.iota(jnp.int32, L)
    zero_idx = jnp.zeros((L,), jnp.int32)

    qbufs = (q_v, q_v1)
    kbufs = (k_v, k_v1)
    arows = (arow_v, arow_v1)
    gqs = (gq0, gq1)
    gks = (gk0, gk1)


    sis = (si0, si1)
    dis = (di0, di1)

    for s in range(2):
        for g in range(CH // L):
            sis[s][pl.ds(g * L, L)] = src_v[s, pl.ds(g * L, L)]
            dis[s][pl.ds(g * L, L)] = dst_v[s, pl.ds(g * L, L)]
        pltpu.async_copy(q_hbm.at[dis[s]], qbufs[s], gqs[s])
        pltpu.async_copy(k_hbm.at[sis[s]], kbufs[s], gks[s])

    @pl.loop(0, NCHUNK // 2)
    def _(i):
        for s in range(2):
            ci = 2 * i + s
            qb, kb, ar = qbufs[s], kbufs[s], arows[s]
            pltpu.make_async_copy(q_hbm.at[dis[s]], qb, gqs[s]).wait()
            pltpu.make_async_copy(k_hbm.at[sis[s]], kb, gks[s]).wait()

            @pl.loop(0, CH // L)
            def _(g):
                avp = [jnp.zeros((L,), jnp.float32) for _ in range(8)]
                for j in range(L):
                    e = g * L + j
                    acc = qb[e, pl.ds(0, L)] * kb[e, pl.ds(0, L)]
                    for h in range(1, C // L):
                        acc = acc + qb[e, pl.ds(h * L, L)] * kb[e, pl.ds(h * L, L)]
                    avp[j % 8] = jnp.where(iota == j, jnp.sum(acc), avp[j % 8])
                av = ((avp[0] + avp[1]) + (avp[2] + avp[3])) + (
                    (avp[4] + avp[5]) + (avp[6] + avp[7]))
                av = jnp.exp(av * INV_SQRT_C)
                a_v[ci, pl.ds(g * L, L)] = av
                plsc.store_scatter(ar, [iota + g * L, zero_idx], av)

            pltpu.sync_copy(ar, dacc.at[sis[s]], add=True)

            cn = jnp.minimum(ci + 2, NCHUNK - 1)
            for g in range(CH // L):
                sis[s][pl.ds(g * L, L)] = src_v[cn, pl.ds(g * L, L)]
                dis[s][pl.ds(g * L, L)] = dst_v[cn, pl.ds(g * L, L)]
            pltpu.async_copy(q_hbm.at[dis[s]], qb, gqs[s])
            pltpu.async_copy(k_hbm.at[sis[s]], kb, gks[s])


    for s in range(2):
        pltpu.make_async_copy(q_hbm.at[dis[s]], qbufs[s], gqs[s]).wait()
        pltpu.make_async_copy(k_hbm.at[sis[s]], kbufs[s], gks[s]).wait()

    pltpu.sync_copy(a_v, alpha_hbm.at[wid])
    plsc.subcore_barrier()
    pltpu.sync_copy(dacc.at[pl.ds(sid * ROWS_PER_SUB, ROWS_PER_SUB)],
                    dpart_hbm.at[pl.ds(cid * NP + sid * ROWS_PER_SUB, ROWS_PER_SUB)])


def _combine_kernel(d_ref, v_ref, o_ref):
    d = d_ref[0:NP, 0:1] + d_ref[NP:2 * NP, 0:1]
    o_ref[...] = v_ref[...] / (d + 1e-16)


def _sc_msg_kernel(vp_hbm, src_hbm, dst_hbm, alpha_hbm, opart_hbm,
                   src_v, dst_v, a_v, v_v0, v_v1, si0, si1, di0, di1, oacc,
                   gv0, gv1):
    cid = lax.axis_index("c")
    sid = lax.axis_index("s")
    wid = sid * NC + cid

    vbufs = (v_v0, v_v1)
    gvs = (gv0, gv1)
    sis = (si0, si1)
    dis = (di0, di1)

    @pl.loop(0, CHB)
    def _(i):
        for g in range(C // L):
            v_v0[i, pl.ds(g * L, L)] = jnp.zeros((L,), jnp.float32)

    @pl.loop(0, ROWS_PER_SUB // CHB)
    def _(b):
        pltpu.sync_copy(v_v0, oacc.at[pl.ds(sid * ROWS_PER_SUB + b * CHB, CHB)])

    plsc.subcore_barrier()

    pltpu.sync_copy(src_hbm.at[wid], src_v)
    pltpu.sync_copy(dst_hbm.at[wid], dst_v)
    pltpu.sync_copy(alpha_hbm.at[wid], a_v)

    for s in range(2):
        for g in range(CHB // L):
            sis[s][pl.ds(g * L, L)] = src_v[s, pl.ds(g * L, L)]
        pltpu.async_copy(vp_hbm.at[sis[s]], vbufs[s], gvs[s])

    @pl.loop(0, NCHB // 2)
    def _(i):
        for s in range(2):
            ci = 2 * i + s
            vb = vbufs[s]
            pltpu.make_async_copy(vp_hbm.at[sis[s]], vb, gvs[s]).wait()

            @pl.loop(0, CHB // L)
            def _(g):
                av = a_v[ci, pl.ds(g * L, L)]
                for j in range(L):
                    e = g * L + j
                    sv = jnp.full((L,), av[j], jnp.float32)
                    for h in range(C // L):
                        vb[e, pl.ds(h * L, L)] = vb[e, pl.ds(h * L, L)] * sv
                dis[s][pl.ds(g * L, L)] = dst_v[ci, pl.ds(g * L, L)]

            pltpu.sync_copy(vb, oacc.at[dis[s]], add=True)

            cn = jnp.minimum(ci + 2, NCHB - 1)
            for g in range(CHB // L):
                sis[s][pl.ds(g * L, L)] = src_v[cn, pl.ds(g * L, L)]
            pltpu.async_copy(vp_hbm.at[sis[s]], vb, gvs[s])

    for s in range(2):
        pltpu.make_async_copy(vp_hbm.at[sis[s]], vbufs[s], gvs[s]).wait()

    plsc.subcore_barrier()
    pltpu.sync_copy(oacc.at[pl.ds(sid * ROWS_PER_SUB, ROWS_PER_SUB)],
                    opart_hbm.at[pl.ds(cid * NP + sid * ROWS_PER_SUB, ROWS_PER_SUB)])


def _final_kernel(op_ref, s_ref, lns_ref, lnb_ref, o_ref):
    o = op_ref[0:N, :] + op_ref[NP:NP + N, :] + s_ref[0:N, :]
    mean = jnp.mean(o, axis=1, keepdims=True)
    cen = o - mean
    var = jnp.mean(cen * cen, axis=1, keepdims=True)
    h = cen * lax.rsqrt(var + 1e-5) * lns_ref[...] + lnb_ref[...]
    o_ref[...] = jnp.where(h > 0, h, jnp.exp(jnp.minimum(h, 0.0)) - 1.0)


def kernel(x, att_index, Wq, bq, Wk, bk, Wv, bv, Ws, bs, ln_scale, ln_bias):
    Bb, Tt, Nn, Cin = x.shape
    x2 = x.reshape(Nn, Cin)
    xp = jnp.zeros((NP, Cin), jnp.float32).at[:Nn].set(x2)

    src = att_index[0].astype(jnp.int32).reshape(NW, E // NW)
    dst = att_index[1].astype(jnp.int32).reshape(NW, E // NW)
    padw = EW - E // NW
    pad = jnp.broadcast_to(N + jnp.arange(padw, dtype=jnp.int32), (NW, padw))
    srcp = jnp.concatenate([src, pad], axis=1).reshape(NW, NCHUNK, CH)
    dstp = jnp.concatenate([dst, pad], axis=1).reshape(NW, NCHUNK, CH)

    q, k, v, s = pl.pallas_call(
        _proj_kernel,
        out_shape=[jax.ShapeDtypeStruct((NP, C), jnp.float32)] * 4,
    )(xp, Wq, bq.reshape(1, C), Wk, bk.reshape(1, C),
      Wv, bv.reshape(1, C), Ws, bs.reshape(1, C))

    alpha, dpart = pl.kernel(
        _sc_alpha_kernel,
        out_type=[jax.ShapeDtypeStruct((NW, NCHUNK, CH), jnp.float32),
                  jax.ShapeDtypeStruct((2 * NP, L), jnp.float32)],
        mesh=_mesh,
        compiler_params=_sc_params,
        scratch_types=[
            pltpu.VMEM((NCHUNK, CH), jnp.int32),
            pltpu.VMEM((NCHUNK, CH), jnp.int32),
            pltpu.VMEM((NCHUNK, CH), jnp.float32),
            pltpu.VMEM((CH, C), jnp.float32),
            pltpu.VMEM((CH, C), jnp.float32),
            pltpu.VMEM((CH, C), jnp.float32),
            pltpu.VMEM((CH, C), jnp.float32),
            pltpu.VMEM((CH, L), jnp.float32),
            pltpu.VMEM((CH, L), jnp.float32),
            pltpu.VMEM((CH,), jnp.int32),
            pltpu.VMEM((CH,), jnp.int32),
            pltpu.VMEM((CH,), jnp.int32),
            pltpu.VMEM((CH,), jnp.int32),
            pltpu.VMEM((ROWS_PER_SUB, L), jnp.float32),
            pltpu.VMEM_SHARED((NP, L), jnp.float32),
            pltpu.SemaphoreType.DMA,
            pltpu.SemaphoreType.DMA,
            pltpu.SemaphoreType.DMA,
            pltpu.SemaphoreType.DMA,
        ],
    )(q, k, srcp, dstp)

    vp = pl.pallas_call(
        _combine_kernel,
        out_shape=jax.ShapeDtypeStruct((NP, C), jnp.float32),
    )(dpart, v)

    opart = pl.kernel(
        _sc_msg_kernel,
        out_type=jax.ShapeDtypeStruct((2 * NP, C), jnp.float32),
        mesh=_mesh,
        compiler_params=_sc_params,
        scratch_types=[
            pltpu.VMEM((NCHB, CHB), jnp.int32),
            pltpu.VMEM((NCHB, CHB), jnp.int32),
            pltpu.VMEM((NCHB, CHB), jnp.float32),
            pltpu.VMEM((CHB, C), jnp.float32),
            pltpu.VMEM((CHB, C), jnp.float32),
            pltpu.VMEM((CHB,), jnp.int32),
            pltpu.VMEM((CHB,), jnp.int32),
            pltpu.VMEM((CHB,), jnp.int32),
            pltpu.VMEM((CHB,), jnp.int32),
            pltpu.VMEM_SHARED((NP, C), jnp.float32),
            pltpu.SemaphoreType.DMA,
            pltpu.SemaphoreType.DMA,
        ],
    )(vp, srcp.reshape(NW, NCHB, CHB), dstp.reshape(NW, NCHB, CHB),
      alpha.reshape(NW, NCHB, CHB))

    out = pl.pallas_call(
        _final_kernel,
        out_shape=jax.ShapeDtypeStruct((N, C), jnp.float32),
    )(opart, s, ln_scale.reshape(1, C), ln_bias.reshape(1, C))

    return out.reshape(Bb, Tt, Nn, C)


# speedup vs baseline: 1.0334x; 1.0334x over previous
import dataclasses
import functools
import math

import jax
import jax.numpy as jnp
from jax import lax
from jax.experimental import pallas as pl
from jax.experimental.pallas import tpu as pltpu
from jax.experimental.pallas import tpu_sc as plsc

N = 10000
E = 320000
C = 128

NC = 2
NS = 16
L = 16
NW = NC * NS

NP = 10240
CH = 128
EP = 327680
EW = EP // NW
NCHUNK = EW // CH
ROWS_PER_SUB = NP // NS
CHB = 64
NCHB = EW // CHB

INV_SQRT_C = 1.0 / math.sqrt(C)

_mesh = plsc.VectorSubcoreMesh(
    core_axis_name="c", subcore_axis_name="s", num_cores=NC, num_subcores=NS)

_sc_params = pltpu.CompilerParams(
    needs_layout_passes=False, use_tc_tiling_on_sc=False)


def _proj_kernel(x_ref, wq_ref, bq_ref, wk_ref, bk_ref, wv_ref, bv_ref,
                 ws_ref, bs_ref, q_ref, k_ref, v_ref, s_ref):
    x = x_ref[...]
    q_ref[...] = jnp.dot(x, wq_ref[...], preferred_element_type=jnp.float32) + bq_ref[...]
    k_ref[...] = jnp.dot(x, wk_ref[...], preferred_element_type=jnp.float32) + bk_ref[...]
    v_ref[...] = jnp.dot(x, wv_ref[...], preferred_element_type=jnp.float32) + bv_ref[...]
    s_ref[...] = jnp.dot(x, ws_ref[...], preferred_element_type=jnp.float32) + bs_ref[...]


def _sc_alpha_kernel(q_hbm, k_hbm, src_hbm, dst_hbm, alpha_hbm, dpart_hbm,
                     src_v, dst_v, a_v, q_v, q_v1, k_v, k_v1, arow_v, arow_v1,
                     si0, si1, di0, di1, z_v, dacc, gq0, gq1, gk0, gk1):
    cid = lax.axis_index("c")
    sid = lax.axis_index("s")
    wid = sid * NC + cid

    @pl.loop(0, CH)
    def _(i):
        arow_v[i] = jnp.zeros((L,), jnp.float32)
        arow_v1[i] = jnp.zeros((L,), jnp.float32)

    @pl.loop(0, ROWS_PER_SUB)
    def _(i):
        z_v[i] = jnp.zeros((L,), jnp.float32)

    pltpu.sync_copy(z_v, dacc.at[pl.ds(sid * ROWS_PER_SUB, ROWS_PER_SUB)])
    plsc.subcore_barrier()

    pltpu.sync_copy(src_hbm.at[wid], src_v)
    pltpu.sync_copy(dst_hbm.at[wid], dst_v)

    iota = lax.iota(jnp.int32, L)
    zero_idx = jnp.zeros((L,), jnp.int32)

    qbufs = (q_v, q_v1)
    kbufs = (k_v, k_v1)
    arows = (arow_v, arow_v1)
    gqs = (gq0, gq1)
    gks = (gk0, gk1)


    sis = (si0, si1)
    dis = (di0, di1)

    for s in range(2):
        for g in range(CH // L):
            sis[s][pl.ds(g * L, L)] = src_v[s, pl.ds(g * L, L)]
            dis[s][pl.ds(g * L, L)] = dst_v[s, pl.ds(g * L, L)]
        pltpu.async_copy(q_hbm.at[dis[s]], qbufs[s], gqs[s])
        pltpu.async_copy(k_hbm.at[sis[s]], kbufs[s], gks[s])

    @pl.loop(0, NCHUNK // 2)
    def _(i):
        for s in range(2):
            ci = 2 * i + s
            qb, kb, ar = qbufs[s], kbufs[s], arows[s]
            pltpu.make_async_copy(q_hbm.at[dis[s]], qb, gqs[s]).wait()
            pltpu.make_async_copy(k_hbm.at[sis[s]], kb, gks[s]).wait()

            @pl.loop(0, CH // L)
            def _(g):
                avp = [jnp.zeros((L,), jnp.float32) for _ in range(4)]
                for j in range(L):
                    e = g * L + j
                    acc = qb[e, pl.ds(0, L)] * kb[e, pl.ds(0, L)]
                    for h in range(1, C // L):
                        acc = acc + qb[e, pl.ds(h * L, L)] * kb[e, pl.ds(h * L, L)]
                    avp[j % 4] = jnp.where(iota == j, jnp.sum(acc), avp[j % 4])
                av = (avp[0] + avp[1]) + (avp[2] + avp[3])
                av = jnp.exp(av * INV_SQRT_C)
                a_v[ci, pl.ds(g * L, L)] = av
                plsc.store_scatter(ar, [iota + g * L, zero_idx], av)

            pltpu.sync_copy(ar, dacc.at[sis[s]], add=True)

            cn = jnp.minimum(ci + 2, NCHUNK - 1)
            for g in range(CH // L):
                sis[s][pl.ds(g * L, L)] = src_v[cn, pl.ds(g * L, L)]
                dis[s][pl.ds(g * L, L)] = dst_v[cn, pl.ds(g * L, L)]
            pltpu.async_copy(q_hbm.at[dis[s]], qb, gqs[s])
            pltpu.async_copy(k_hbm.at[sis[s]], kb, gks[s])


    for s in range(2):
        pltpu.make_async_copy(q_hbm.at[dis[s]], qbufs[s], gqs[s]).wait()
        pltpu.make_async_copy(k_hbm.at[sis[s]], kbufs[s], gks[s]).wait()

    pltpu.sync_copy(a_v, alpha_hbm.at[wid])
    plsc.subcore_barrier()
    pltpu.sync_copy(dacc.at[pl.ds(sid * ROWS_PER_SUB, ROWS_PER_SUB)],
                    dpart_hbm.at[pl.ds(cid * NP + sid * ROWS_PER_SUB, ROWS_PER_SUB)])


def _combine_kernel(d_ref, v_ref, o_ref):
    d = d_ref[0:NP, 0:1] + d_ref[NP:2 * NP, 0:1]
    o_ref[...] = v_ref[...] / (d + 1e-16)


def _sc_msg_kernel(vp_hbm, src_hbm, dst_hbm, alpha_hbm, opart_hbm,
                   src_v, dst_v, a_v, v_v0, v_v1, si0, si1, di0, di1, oacc,
                   gv0, gv1):
    cid = lax.axis_index("c")
    sid = lax.axis_index("s")
    wid = sid * NC + cid

    vbufs = (v_v0, v_v1)
    gvs = (gv0, gv1)
    sis = (si0, si1)
    dis = (di0, di1)

    @pl.loop(0, CHB)
    def _(i):
        for g in range(C // L):
            v_v0[i, pl.ds(g * L, L)] = jnp.zeros((L,), jnp.float32)

    @pl.loop(0, ROWS_PER_SUB // CHB)
    def _(b):
        pltpu.sync_copy(v_v0, oacc.at[pl.ds(sid * ROWS_PER_SUB + b * CHB, CHB)])

    plsc.subcore_barrier()

    pltpu.sync_copy(src_hbm.at[wid], src_v)
    pltpu.sync_copy(dst_hbm.at[wid], dst_v)
    pltpu.sync_copy(alpha_hbm.at[wid], a_v)

    for s in range(2):
        for g in range(CHB // L):
            sis[s][pl.ds(g * L, L)] = src_v[s, pl.ds(g * L, L)]
        pltpu.async_copy(vp_hbm.at[sis[s]], vbufs[s], gvs[s])

    @pl.loop(0, NCHB // 2)
    def _(i):
        for s in range(2):
            ci = 2 * i + s
            vb = vbufs[s]
            pltpu.make_async_copy(vp_hbm.at[sis[s]], vb, gvs[s]).wait()

            @pl.loop(0, CHB // L)
            def _(g):
                av = a_v[ci, pl.ds(g * L, L)]
                for j in range(L):
                    e = g * L + j
                    sv = jnp.full((L,), av[j], jnp.float32)
                    for h in range(C // L):
                        vb[e, pl.ds(h * L, L)] = vb[e, pl.ds(h * L, L)] * sv
                dis[s][pl.ds(g * L, L)] = dst_v[ci, pl.ds(g * L, L)]

            pltpu.sync_copy(vb, oacc.at[dis[s]], add=True)

            cn = jnp.minimum(ci + 2, NCHB - 1)
            for g in range(CHB // L):
                sis[s][pl.ds(g * L, L)] = src_v[cn, pl.ds(g * L, L)]
            pltpu.async_copy(vp_hbm.at[sis[s]], vb, gvs[s])

    for s in range(2):
        pltpu.make_async_copy(vp_hbm.at[sis[s]], vbufs[s], gvs[s]).wait()

    plsc.subcore_barrier()
    pltpu.sync_copy(oacc.at[pl.ds(sid * ROWS_PER_SUB, ROWS_PER_SUB)],
                    opart_hbm.at[pl.ds(cid * NP + sid * ROWS_PER_SUB, ROWS_PER_SUB)])


def _final_kernel(op_ref, s_ref, lns_ref, lnb_ref, o_ref):
    o = op_ref[0:N, :] + op_ref[NP:NP + N, :] + s_ref[0:N, :]
    mean = jnp.mean(o, axis=1, keepdims=True)
    cen = o - mean
    var = jnp.mean(cen * cen, axis=1, keepdims=True)
    h = cen * lax.rsqrt(var + 1e-5) * lns_ref[...] + lnb_ref[...]
    o_ref[...] = jnp.where(h > 0, h, jnp.exp(jnp.minimum(h, 0.0)) - 1.0)


def kernel(x, att_index, Wq, bq, Wk, bk, Wv, bv, Ws, bs, ln_scale, ln_bias):
    Bb, Tt, Nn, Cin = x.shape
    x2 = x.reshape(Nn, Cin)
    xp = jnp.zeros((NP, Cin), jnp.float32).at[:Nn].set(x2)

    src = att_index[0].astype(jnp.int32).reshape(NW, E // NW)
    dst = att_index[1].astype(jnp.int32).reshape(NW, E // NW)
    padw = EW - E // NW
    pad = jnp.broadcast_to(N + jnp.arange(padw, dtype=jnp.int32), (NW, padw))
    srcp = jnp.concatenate([src, pad], axis=1).reshape(NW, NCHUNK, CH)
    dstp = jnp.concatenate([dst, pad], axis=1).reshape(NW, NCHUNK, CH)

    q, k, v, s = pl.pallas_call(
        _proj_kernel,
        out_shape=[jax.ShapeDtypeStruct((NP, C), jnp.float32)] * 4,
    )(xp, Wq, bq.reshape(1, C), Wk, bk.reshape(1, C),
      Wv, bv.reshape(1, C), Ws, bs.reshape(1, C))

    alpha, dpart = pl.kernel(
        _sc_alpha_kernel,
        out_type=[jax.ShapeDtypeStruct((NW, NCHUNK, CH), jnp.float32),
                  jax.ShapeDtypeStruct((2 * NP, L), jnp.float32)],
        mesh=_mesh,
        compiler_params=_sc_params,
        scratch_types=[
            pltpu.VMEM((NCHUNK, CH), jnp.int32),
            pltpu.VMEM((NCHUNK, CH), jnp.int32),
            pltpu.VMEM((NCHUNK, CH), jnp.float32),
            pltpu.VMEM((CH, C), jnp.float32),
            pltpu.VMEM((CH, C), jnp.float32),
            pltpu.VMEM((CH, C), jnp.float32),
            pltpu.VMEM((CH, C), jnp.float32),
            pltpu.VMEM((CH, L), jnp.float32),
            pltpu.VMEM((CH, L), jnp.float32),
            pltpu.VMEM((CH,), jnp.int32),
            pltpu.VMEM((CH,), jnp.int32),
            pltpu.VMEM((CH,), jnp.int32),
            pltpu.VMEM((CH,), jnp.int32),
            pltpu.VMEM((ROWS_PER_SUB, L), jnp.float32),
            pltpu.VMEM_SHARED((NP, L), jnp.float32),
            pltpu.SemaphoreType.DMA,
            pltpu.SemaphoreType.DMA,
            pltpu.SemaphoreType.DMA,
            pltpu.SemaphoreType.DMA,
        ],
    )(q, k, srcp, dstp)

    vp = pl.pallas_call(
        _combine_kernel,
        out_shape=jax.ShapeDtypeStruct((NP, C), jnp.float32),
    )(dpart, v)

    opart = pl.kernel(
        _sc_msg_kernel,
        out_type=jax.ShapeDtypeStruct((2 * NP, C), jnp.float32),
        mesh=_mesh,
        compiler_params=_sc_params,
        scratch_types=[
            pltpu.VMEM((NCHB, CHB), jnp.int32),
            pltpu.VMEM((NCHB, CHB), jnp.int32),
            pltpu.VMEM((NCHB, CHB), jnp.float32),
            pltpu.VMEM((CHB, C), jnp.float32),
            pltpu.VMEM((CHB, C), jnp.float32),
            pltpu.VMEM((CHB,), jnp.int32),
            pltpu.VMEM((CHB,), jnp.int32),
            pltpu.VMEM((CHB,), jnp.int32),
            pltpu.VMEM((CHB,), jnp.int32),
            pltpu.VMEM_SHARED((NP, C), jnp.float32),
            pltpu.SemaphoreType.DMA,
            pltpu.SemaphoreType.DMA,
        ],
    )(vp, srcp.reshape(NW, NCHB, CHB), dstp.reshape(NW, NCHB, CHB),
      alpha.reshape(NW, NCHB, CHB))

    out = pl.pallas_call(
        _final_kernel,
        out_shape=jax.ShapeDtypeStruct((N, C), jnp.float32),
    )(opart, s, ln_scale.reshape(1, C), ln_bias.reshape(1, C))

    return out.reshape(Bb, Tt, Nn, C)
